# Initial kernel scaffold; baseline (speedup 1.0000x reference)
import jax
import jax.numpy as jnp
from jax.experimental import pallas as pl


def kernel(ids, table):
    raise NotImplementedError("write your pallas kernel here")


# speedup vs baseline: 2.8612x; 2.8612x over previous
import functools

import jax
import jax.numpy as jnp
from jax import lax
from jax.experimental import pallas as pl
from jax.experimental.pallas import tpu as pltpu
from jax.experimental.pallas import tpu_sc as plsc

B = 16384
L = 50
D = 32
NW = 32
RPW = B * L // NW
EPC = 2
CH = EPC * L
NCH = RPW // CH
EPW = B // NW
NBUF = 4
LANES = 16


def _sc_lookup(ids3, table):
    mesh = plsc.VectorSubcoreMesh(core_axis_name="c", subcore_axis_name="s")

    @functools.partial(
        pl.kernel,
        mesh=mesh,
        out_type=jax.ShapeDtypeStruct((B, D), jnp.float32),
        compiler_params=pltpu.CompilerParams(use_tc_tiling_on_sc=False),
        scratch_types=[
            pltpu.VMEM((NCH, CH), jnp.int32),
            pltpu.VMEM((EPW, D), jnp.float32),
        ]
        + [pltpu.VMEM((CH, D), jnp.float32) for _ in range(NBUF)]
        + [pltpu.SemaphoreType.DMA for _ in range(NBUF)],
    )
    def k(ids_hbm, table_hbm, out_hbm, idx_v, out_v, *bufs_sems):
        bufs = bufs_sems[:NBUF]
        sems = bufs_sems[NBUF:]
        wid = lax.axis_index("s") * 2 + lax.axis_index("c")
        pltpu.sync_copy(ids_hbm.at[wid], idx_v)

        def issue(c, b):
            pltpu.async_copy(table_hbm.at[idx_v.at[c]], bufs[b], sems[b])

        for b in range(NBUF):
            issue(b, b)

        inv = jnp.float32(1.0 / L)

        def group_body(g, carry):
            for b in range(NBUF):
                c = g * NBUF + b
                pltpu.make_async_copy(
                    table_hbm.at[idx_v.at[c]], bufs[b], sems[b]
                ).wait()
                for k_e in range(EPC):
                    acc = [bufs[b][k_e * L, pl.ds(h * LANES, LANES)]
                           for h in range(D // LANES)]
                    for l_t in range(1, L):
                        for h in range(D // LANES):
                            acc[h] = acc[h] + bufs[b][
                                k_e * L + l_t, pl.ds(h * LANES, LANES)]
                    e = c * EPC + k_e
                    for h in range(D // LANES):
                        out_v[e, pl.ds(h * LANES, LANES)] = acc[h] * inv
                nxt = c + NBUF

                @pl.when(nxt < NCH)
                def _():
                    issue(nxt, b)
            return carry

        lax.fori_loop(0, NCH // NBUF, group_body, 0)
        pltpu.sync_copy(out_v, out_hbm.at[pl.ds(wid * EPW, EPW)])

    return k(ids3, table)


def kernel(ids, table):
    ids3 = ids.astype(jnp.int32).reshape(NW, NCH, CH)
    return _sc_lookup(ids3, table)
